# baseline (speedup 1.0000x reference)
import functools

import jax
import jax.numpy as jnp
from jax import lax
from jax.experimental import pallas as pl
from jax.experimental.pallas import tpu as pltpu
from jax.experimental.pallas import tpu_sc as plsc

N_NODES = 10000
N_EDGES = 320000
D = 128

_NC = 2
_NS = 16
_NW = _NC * _NS

_SLICES = 2
_ES = N_EDGES // _SLICES
_EW = _ES // _NW
_K = 200
_CHUNKS = _EW // _K

_BN = 2000
_BE = 6400


def _proj_body(node_ref, w_ref, out_ref):
    out_ref[0] = jnp.dot(node_ref[...], w_ref[0],
                         preferred_element_type=jnp.float32)


def _project(node_features, w_pair):
    return pl.pallas_call(
        _proj_body,
        grid=(2, N_NODES // _BN),
        in_specs=[
            pl.BlockSpec((_BN, D), lambda i, j: (j, 0)),
            pl.BlockSpec((1, D, D), lambda i, j: (i, 0, 0)),
        ],
        out_specs=pl.BlockSpec((1, _BN, D), lambda i, j: (i, j, 0)),
        out_shape=jax.ShapeDtypeStruct((2, N_NODES, D), jnp.float32),
    )(node_features, w_pair)


def _gather_body(table_hbm, idx_hbm, out_hbm, is0, is1, id0, id1,
                 rs0, rs1, sg0, sg1, sw0, sw1):
    wid = lax.axis_index("s") * _NC + lax.axis_index("c")
    base = wid * _EW

    idx_s = (is0, is1)
    idx_d = (id0, id1)
    rows = (rs0, rs1)
    sg = (sg0, sg1)
    sw = (sw0, sw1)
    pend_w = [None, None]
    pend_g = [None, None]

    def start_chunk(j, b):
        off = base + j * _K
        pltpu.sync_copy(idx_hbm.at[pl.ds(off, _K)], idx_s[b])
        pltpu.sync_copy(idx_hbm.at[pl.ds(_ES + off, _K)], idx_d[b])
        pend_g[b] = pltpu.async_copy(table_hbm.at[idx_s[b]], rows[b], sg[b])

    start_chunk(0, 0)
    for j in range(_CHUNKS):
        b = j & 1
        nb = 1 - b
        if j + 1 < _CHUNKS:
            if pend_w[nb] is not None:
                pend_w[nb].wait()
            start_chunk(j + 1, nb)
        pend_g[b].wait()
        g1 = pltpu.async_copy(table_hbm.at[idx_d[b]], rows[b], sg[b], add=True)
        g1.wait()
        pend_w[b] = pltpu.async_copy(rows[b],
                                     out_hbm.at[pl.ds(base + j * _K, _K)],
                                     sw[b])
    pend_w[_CHUNKS & 1].wait()
    pend_w[1 - (_CHUNKS & 1)].wait()


def _gather(table, idx_slice):
    mesh = plsc.VectorSubcoreMesh(core_axis_name="c", subcore_axis_name="s",
                                  num_cores=_NC, num_subcores=_NS)
    f = functools.partial(
        pl.kernel,
        mesh=mesh,
        out_type=jax.ShapeDtypeStruct((_ES, D), jnp.float32),
        scratch_types=[
            pltpu.VMEM((_K,), jnp.int32),
            pltpu.VMEM((_K,), jnp.int32),
            pltpu.VMEM((_K,), jnp.int32),
            pltpu.VMEM((_K,), jnp.int32),
            pltpu.VMEM((_K, D), jnp.float32),
            pltpu.VMEM((_K, D), jnp.float32),
            pltpu.SemaphoreType.DMA,
            pltpu.SemaphoreType.DMA,
            pltpu.SemaphoreType.DMA,
            pltpu.SemaphoreType.DMA,
        ],
    )(_gather_body)
    return f(table, idx_slice)


def _mlp_body(s_ref, spt_ref, w1sp_ref, b1_ref, w2_ref, b2_ref,
              w3_ref, b3_ref, out_ref):
    x = s_ref[...]
    x = x + lax.dot_general(spt_ref[...], w1sp_ref[...],
                            (((0,), (0,)), ((), ())),
                            preferred_element_type=jnp.float32) + b1_ref[...]
    h = jnp.maximum(x, 0.0)
    h = jnp.dot(h, w2_ref[...], preferred_element_type=jnp.float32) + b2_ref[...]
    h = jnp.maximum(h, 0.0)
    out_ref[...] = lax.dot_general(w3_ref[...], h,
                                   (((0,), (1,)), ((), ())),
                                   preferred_element_type=jnp.float32) + b3_ref[...]


def _mlp(s, spatial_t, w1sp, b1, w2, b2, w3, b3):
    nblk = _ES // _BE
    d2, d3 = w2.shape[1], w3.shape[1]
    return pl.pallas_call(
        _mlp_body,
        grid=(nblk,),
        in_specs=[
            pl.BlockSpec((_BE, D), lambda i: (i, 0)),
            pl.BlockSpec((2, _BE), lambda i: (0, i)),
            pl.BlockSpec((2, D), lambda i: (0, 0)),
            pl.BlockSpec((1, D), lambda i: (0, 0)),
            pl.BlockSpec((D, d2), lambda i: (0, 0)),
            pl.BlockSpec((1, d2), lambda i: (0, 0)),
            pl.BlockSpec((d2, d3), lambda i: (0, 0)),
            pl.BlockSpec((d3, 1), lambda i: (0, 0)),
        ],
        out_specs=pl.BlockSpec((d3, _BE), lambda i: (0, i)),
        out_shape=jax.ShapeDtypeStruct((d3, _ES), jnp.float32),
    )(s, spatial_t, w1sp, b1, w2, b2, w3, b3)


def kernel(node_features, edge_index, spatial_features, W1, b1, W2, b2, W3, b3):
    edge_index = edge_index.astype(jnp.int32)

    w_pair = W1[:2 * D].reshape(2, D, D)
    table = _project(node_features, w_pair).reshape(2 * N_NODES, D)

    spatial_t = spatial_features.T
    w1sp = W1[2 * D:]
    b1r = b1.reshape(1, D)
    b2r = b2.reshape(1, -1)
    b3r = b3.reshape(-1, 1)

    outs = []
    for i in range(_SLICES):
        sl = slice(i * _ES, (i + 1) * _ES)
        idx_slice = jnp.concatenate(
            [edge_index[0, sl], edge_index[1, sl] + N_NODES])
        s = _gather(table, idx_slice)
        outs.append(_mlp(s, spatial_t[:, sl], w1sp, b1r, W2, b2r, W3, b3r))

    out_t = jnp.concatenate(outs, axis=1) if _SLICES > 1 else outs[0]
    return out_t.T


# speedup vs baseline: 1.1394x; 1.1394x over previous
import functools

import jax
import jax.numpy as jnp
from jax import lax
from jax.experimental import pallas as pl
from jax.experimental.pallas import tpu as pltpu
from jax.experimental.pallas import tpu_sc as plsc

N_NODES = 10000
N_EDGES = 320000
D = 128

_NC = 2
_NS = 16
_NW = _NC * _NS

_SLICES = 2
_ES = N_EDGES // _SLICES
_EW = _ES // _NW
_K = 200
_CHUNKS = _EW // _K

_BN = 2000
_BE = 6400


def _proj_body(node_ref, w_ref, out_ref):
    out_ref[0] = jnp.dot(node_ref[...], w_ref[0],
                         preferred_element_type=jnp.float32)


def _project(node_features, w_pair):
    return pl.pallas_call(
        _proj_body,
        grid=(2, N_NODES // _BN),
        in_specs=[
            pl.BlockSpec((_BN, D), lambda i, j: (j, 0)),
            pl.BlockSpec((1, D, D), lambda i, j: (i, 0, 0)),
        ],
        out_specs=pl.BlockSpec((1, _BN, D), lambda i, j: (i, j, 0)),
        out_shape=jax.ShapeDtypeStruct((2, N_NODES, D), jnp.float32),
    )(node_features, w_pair)


def _chunk_add(rows_s, rows_d):
    def row(e, carry):
        for c in range(D // 16):
            sl = pl.ds(c * 16, 16)
            rows_s[e, sl] = rows_s[e, sl] + rows_d[e, sl]
        return carry

    lax.fori_loop(0, _K, row, 0)


def _gather_body(table_hbm, idx_hbm, out_hbm, is0, is1, id0, id1,
                 rs0, rs1, rd0, rd1, sg0, sg1, sw0, sw1):
    wid = lax.axis_index("s") * _NC + lax.axis_index("c")
    base = wid * _EW

    idx_s = (is0, is1)
    idx_d = (id0, id1)
    rows_s = (rs0, rs1)
    rows_d = (rd0, rd1)
    sg = (sg0, sg1)
    sw = (sw0, sw1)
    pend_w = [None, None]
    pend_g = [None, None]

    def start_chunk(j, b):
        off = base + j * _K
        pltpu.sync_copy(idx_hbm.at[pl.ds(off, _K)], idx_s[b])
        pltpu.sync_copy(idx_hbm.at[pl.ds(_ES + off, _K)], idx_d[b])
        g0 = pltpu.async_copy(table_hbm.at[idx_s[b]], rows_s[b], sg[b])
        g1 = pltpu.async_copy(table_hbm.at[idx_d[b]], rows_d[b], sg[b])
        pend_g[b] = (g0, g1)

    start_chunk(0, 0)
    for j in range(_CHUNKS):
        b = j & 1
        nb = 1 - b
        if j + 1 < _CHUNKS:
            if pend_w[nb] is not None:
                pend_w[nb].wait()
            start_chunk(j + 1, nb)
        pend_g[b][0].wait()
        pend_g[b][1].wait()
        _chunk_add(rows_s[b], rows_d[b])
        pend_w[b] = pltpu.async_copy(rows_s[b],
                                     out_hbm.at[pl.ds(base + j * _K, _K)],
                                     sw[b])
    pend_w[_CHUNKS & 1].wait()
    pend_w[1 - (_CHUNKS & 1)].wait()


def _gather(table, idx_slice):
    mesh = plsc.VectorSubcoreMesh(core_axis_name="c", subcore_axis_name="s",
                                  num_cores=_NC, num_subcores=_NS)
    f = functools.partial(
        pl.kernel,
        mesh=mesh,
        out_type=jax.ShapeDtypeStruct((_ES, D), jnp.float32),
        scratch_types=[
            pltpu.VMEM((_K,), jnp.int32),
            pltpu.VMEM((_K,), jnp.int32),
            pltpu.VMEM((_K,), jnp.int32),
            pltpu.VMEM((_K,), jnp.int32),
            pltpu.VMEM((_K, D), jnp.float32),
            pltpu.VMEM((_K, D), jnp.float32),
            pltpu.VMEM((_K, D), jnp.float32),
            pltpu.VMEM((_K, D), jnp.float32),
            pltpu.SemaphoreType.DMA,
            pltpu.SemaphoreType.DMA,
            pltpu.SemaphoreType.DMA,
            pltpu.SemaphoreType.DMA,
        ],
    )(_gather_body)
    return f(table, idx_slice)


def _mlp_body(s_ref, spt_ref, w1sp_ref, b1_ref, w2_ref, b2_ref,
              w3_ref, b3_ref, out_ref):
    x = s_ref[...]
    x = x + lax.dot_general(spt_ref[...], w1sp_ref[...],
                            (((0,), (0,)), ((), ())),
                            preferred_element_type=jnp.float32) + b1_ref[...]
    h = jnp.maximum(x, 0.0)
    h = jnp.dot(h, w2_ref[...], preferred_element_type=jnp.float32) + b2_ref[...]
    h = jnp.maximum(h, 0.0)
    out_ref[...] = lax.dot_general(w3_ref[...], h,
                                   (((0,), (1,)), ((), ())),
                                   preferred_element_type=jnp.float32) + b3_ref[...]


def _mlp(s, spatial_t, w1sp, b1, w2, b2, w3, b3):
    nblk = _ES // _BE
    d2, d3 = w2.shape[1], w3.shape[1]
    return pl.pallas_call(
        _mlp_body,
        grid=(nblk,),
        in_specs=[
            pl.BlockSpec((_BE, D), lambda i: (i, 0)),
            pl.BlockSpec((2, _BE), lambda i: (0, i)),
            pl.BlockSpec((2, D), lambda i: (0, 0)),
            pl.BlockSpec((1, D), lambda i: (0, 0)),
            pl.BlockSpec((D, d2), lambda i: (0, 0)),
            pl.BlockSpec((1, d2), lambda i: (0, 0)),
            pl.BlockSpec((d2, d3), lambda i: (0, 0)),
            pl.BlockSpec((d3, 1), lambda i: (0, 0)),
        ],
        out_specs=pl.BlockSpec((d3, _BE), lambda i: (0, i)),
        out_shape=jax.ShapeDtypeStruct((d3, _ES), jnp.float32),
    )(s, spatial_t, w1sp, b1, w2, b2, w3, b3)


def kernel(node_features, edge_index, spatial_features, W1, b1, W2, b2, W3, b3):
    edge_index = edge_index.astype(jnp.int32)

    w_pair = W1[:2 * D].reshape(2, D, D)
    table = _project(node_features, w_pair).reshape(2 * N_NODES, D)

    spatial_t = spatial_features.T
    w1sp = W1[2 * D:]
    b1r = b1.reshape(1, D)
    b2r = b2.reshape(1, -1)
    b3r = b3.reshape(-1, 1)

    outs = []
    for i in range(_SLICES):
        sl = slice(i * _ES, (i + 1) * _ES)
        idx_slice = jnp.concatenate(
            [edge_index[0, sl], edge_index[1, sl] + N_NODES])
        s = _gather(table, idx_slice)
        outs.append(_mlp(s, spatial_t[:, sl], w1sp, b1r, W2, b2r, W3, b3r))

    out_t = jnp.concatenate(outs, axis=1) if _SLICES > 1 else outs[0]
    return out_t.T


# speedup vs baseline: 1.1395x; 1.0001x over previous
import functools

import jax
import jax.numpy as jnp
from jax import lax
from jax.experimental import pallas as pl
from jax.experimental.pallas import tpu as pltpu
from jax.experimental.pallas import tpu_sc as plsc

N_NODES = 10000
N_EDGES = 320000
D = 128

_NC = 2
_NS = 16
_NW = _NC * _NS

_SLICES = 5
_ES = N_EDGES // _SLICES
_EW = _ES // _NW
_K = 200
_CHUNKS = _EW // _K

_BN = 2000
_BE = 6400


def _proj_body(node_ref, w_ref, out_ref):
    out_ref[0] = jnp.dot(node_ref[...], w_ref[0],
                         preferred_element_type=jnp.float32)


def _project(node_features, w_pair):
    return pl.pallas_call(
        _proj_body,
        grid=(2, N_NODES // _BN),
        in_specs=[
            pl.BlockSpec((_BN, D), lambda i, j: (j, 0)),
            pl.BlockSpec((1, D, D), lambda i, j: (i, 0, 0)),
        ],
        out_specs=pl.BlockSpec((1, _BN, D), lambda i, j: (i, j, 0)),
        out_shape=jax.ShapeDtypeStruct((2, N_NODES, D), jnp.float32),
    )(node_features, w_pair)


def _chunk_add(rows_s, rows_d):
    def row(e, carry):
        for c in range(D // 16):
            sl = pl.ds(c * 16, 16)
            rows_s[e, sl] = rows_s[e, sl] + rows_d[e, sl]
        return carry

    lax.fori_loop(0, _K, row, 0)


def _gather_body(table_hbm, idx_hbm, out_hbm, is0, is1, id0, id1,
                 rs0, rs1, rd0, rd1, sg0, sg1, sw0, sw1):
    wid = lax.axis_index("s") * _NC + lax.axis_index("c")
    base = wid * _EW

    idx_s = (is0, is1)
    idx_d = (id0, id1)
    rows_s = (rs0, rs1)
    rows_d = (rd0, rd1)
    sg = (sg0, sg1)
    sw = (sw0, sw1)
    pend_w = [None, None]
    pend_g = [None, None]

    def start_chunk(j, b):
        off = base + j * _K
        pltpu.sync_copy(idx_hbm.at[pl.ds(off, _K)], idx_s[b])
        pltpu.sync_copy(idx_hbm.at[pl.ds(_ES + off, _K)], idx_d[b])
        g0 = pltpu.async_copy(table_hbm.at[idx_s[b]], rows_s[b], sg[b])
        g1 = pltpu.async_copy(table_hbm.at[idx_d[b]], rows_d[b], sg[b])
        pend_g[b] = (g0, g1)

    start_chunk(0, 0)
    for j in range(_CHUNKS):
        b = j & 1
        nb = 1 - b
        if j + 1 < _CHUNKS:
            if pend_w[nb] is not None:
                pend_w[nb].wait()
            start_chunk(j + 1, nb)
        pend_g[b][0].wait()
        pend_g[b][1].wait()
        _chunk_add(rows_s[b], rows_d[b])
        pend_w[b] = pltpu.async_copy(rows_s[b],
                                     out_hbm.at[pl.ds(base + j * _K, _K)],
                                     sw[b])
    pend_w[_CHUNKS & 1].wait()
    pend_w[1 - (_CHUNKS & 1)].wait()


def _gather(table, idx_slice):
    mesh = plsc.VectorSubcoreMesh(core_axis_name="c", subcore_axis_name="s",
                                  num_cores=_NC, num_subcores=_NS)
    f = functools.partial(
        pl.kernel,
        mesh=mesh,
        out_type=jax.ShapeDtypeStruct((_ES, D), jnp.float32),
        scratch_types=[
            pltpu.VMEM((_K,), jnp.int32),
            pltpu.VMEM((_K,), jnp.int32),
            pltpu.VMEM((_K,), jnp.int32),
            pltpu.VMEM((_K,), jnp.int32),
            pltpu.VMEM((_K, D), jnp.float32),
            pltpu.VMEM((_K, D), jnp.float32),
            pltpu.VMEM((_K, D), jnp.float32),
            pltpu.VMEM((_K, D), jnp.float32),
            pltpu.SemaphoreType.DMA,
            pltpu.SemaphoreType.DMA,
            pltpu.SemaphoreType.DMA,
            pltpu.SemaphoreType.DMA,
        ],
    )(_gather_body)
    return f(table, idx_slice)


def _mlp_body(s_ref, spt_ref, w1sp_ref, b1_ref, w2_ref, b2_ref,
              w3_ref, b3_ref, out_ref):
    x = s_ref[...]
    x = x + lax.dot_general(spt_ref[...], w1sp_ref[...],
                            (((0,), (0,)), ((), ())),
                            preferred_element_type=jnp.float32) + b1_ref[...]
    h = jnp.maximum(x, 0.0)
    h = jnp.dot(h, w2_ref[...], preferred_element_type=jnp.float32) + b2_ref[...]
    h = jnp.maximum(h, 0.0)
    out_ref[...] = lax.dot_general(w3_ref[...], h,
                                   (((0,), (1,)), ((), ())),
                                   preferred_element_type=jnp.float32) + b3_ref[...]


def _mlp(s, spatial_t, w1sp, b1, w2, b2, w3, b3):
    nblk = _ES // _BE
    d2, d3 = w2.shape[1], w3.shape[1]
    return pl.pallas_call(
        _mlp_body,
        grid=(nblk,),
        in_specs=[
            pl.BlockSpec((_BE, D), lambda i: (i, 0)),
            pl.BlockSpec((2, _BE), lambda i: (0, i)),
            pl.BlockSpec((2, D), lambda i: (0, 0)),
            pl.BlockSpec((1, D), lambda i: (0, 0)),
            pl.BlockSpec((D, d2), lambda i: (0, 0)),
            pl.BlockSpec((1, d2), lambda i: (0, 0)),
            pl.BlockSpec((d2, d3), lambda i: (0, 0)),
            pl.BlockSpec((d3, 1), lambda i: (0, 0)),
        ],
        out_specs=pl.BlockSpec((d3, _BE), lambda i: (0, i)),
        out_shape=jax.ShapeDtypeStruct((d3, _ES), jnp.float32),
    )(s, spatial_t, w1sp, b1, w2, b2, w3, b3)


def kernel(node_features, edge_index, spatial_features, W1, b1, W2, b2, W3, b3):
    edge_index = edge_index.astype(jnp.int32)

    w_pair = W1[:2 * D].reshape(2, D, D)
    table = _project(node_features, w_pair).reshape(2 * N_NODES, D)

    spatial_t = spatial_features.T
    w1sp = W1[2 * D:]
    b1r = b1.reshape(1, D)
    b2r = b2.reshape(1, -1)
    b3r = b3.reshape(-1, 1)

    outs = []
    for i in range(_SLICES):
        sl = slice(i * _ES, (i + 1) * _ES)
        idx_slice = jnp.concatenate(
            [edge_index[0, sl], edge_index[1, sl] + N_NODES])
        s = _gather(table, idx_slice)
        outs.append(_mlp(s, spatial_t[:, sl], w1sp, b1r, W2, b2r, W3, b3r))

    out_t = jnp.concatenate(outs, axis=1) if _SLICES > 1 else outs[0]
    return out_t.T
